# Initial kernel scaffold; baseline (speedup 1.0000x reference)
import jax
import jax.numpy as jnp
from jax.experimental import pallas as pl


def kernel(x, edge_index, W1, b1, W2, b2):
    raise NotImplementedError("write your pallas kernel here")


# speedup vs baseline: 1617.9513x; 1617.9513x over previous
import jax
import jax.numpy as jnp
from jax.experimental import pallas as pl


def _mlp_logsoftmax_kernel(x_ref, w1_ref, b1_ref, w2_ref, b2_ref, o_ref):
    x = x_ref[...]
    h = jax.lax.dot_general(
        x, w1_ref[...], (((1,), (1,)), ((), ())),
        preferred_element_type=jnp.float32)
    h = jnp.maximum(h + b1_ref[...], 0.0)
    out = jax.lax.dot_general(
        h, w2_ref[...], (((1,), (1,)), ((), ())),
        preferred_element_type=jnp.float32)
    out = out + b2_ref[...]
    m = jnp.max(out, axis=1, keepdims=True)
    lse = m + jnp.log(jnp.sum(jnp.exp(out - m), axis=1, keepdims=True))
    o_ref[...] = out - lse


def kernel(x, edge_index, W1, b1, W2, b2):
    del edge_index
    n, f_in = x.shape
    hid = W1.shape[0]
    c = W2.shape[0]

    blk = 1000 if n % 1000 == 0 else n

    return pl.pallas_call(
        _mlp_logsoftmax_kernel,
        grid=(n // blk,),
        in_specs=[
            pl.BlockSpec((blk, f_in), lambda i: (i, 0)),
            pl.BlockSpec((hid, f_in), lambda i: (0, 0)),
            pl.BlockSpec((1, hid), lambda i: (0, 0)),
            pl.BlockSpec((c, hid), lambda i: (0, 0)),
            pl.BlockSpec((1, c), lambda i: (0, 0)),
        ],
        out_specs=pl.BlockSpec((blk, c), lambda i: (i, 0)),
        out_shape=jax.ShapeDtypeStruct((n, c), jnp.float32),
    )(x, W1, b1.reshape(1, hid), W2, b2.reshape(1, c))


# speedup vs baseline: 1941.7364x; 1.2001x over previous
import jax
import jax.numpy as jnp
from jax.experimental import pallas as pl


def _mlp_logsoftmax_kernel(x_ref, w1_ref, b1_ref, w2_ref, b2_ref, o_ref):
    x = x_ref[...]
    h = jax.lax.dot_general(
        x, w1_ref[...], (((1,), (1,)), ((), ())),
        preferred_element_type=jnp.float32)
    h = jnp.maximum(h + b1_ref[...], 0.0)
    out = jax.lax.dot_general(
        h, w2_ref[...], (((1,), (1,)), ((), ())),
        preferred_element_type=jnp.float32)
    out = out + b2_ref[...]
    m = jnp.max(out, axis=1, keepdims=True)
    lse = m + jnp.log(jnp.sum(jnp.exp(out - m), axis=1, keepdims=True))
    o_ref[...] = out - lse


def kernel(x, edge_index, W1, b1, W2, b2):
    del edge_index
    n, f_in = x.shape
    hid = W1.shape[0]
    c = W2.shape[0]

    blk = 2000 if n % 2000 == 0 else n

    return pl.pallas_call(
        _mlp_logsoftmax_kernel,
        grid=(n // blk,),
        in_specs=[
            pl.BlockSpec((blk, f_in), lambda i: (i, 0)),
            pl.BlockSpec((hid, f_in), lambda i: (0, 0)),
            pl.BlockSpec((1, hid), lambda i: (0, 0)),
            pl.BlockSpec((c, hid), lambda i: (0, 0)),
            pl.BlockSpec((1, c), lambda i: (0, 0)),
        ],
        out_specs=pl.BlockSpec((blk, c), lambda i: (i, 0)),
        out_shape=jax.ShapeDtypeStruct((n, c), jnp.float32),
    )(x, W1, b1.reshape(1, hid), W2, b2.reshape(1, c))


# speedup vs baseline: 2197.7757x; 1.1319x over previous
import jax
import jax.numpy as jnp
from jax.experimental import pallas as pl


def _mlp_logsoftmax_kernel(x_ref, w1_ref, b1_ref, w2_ref, b2_ref, o_ref):
    x = x_ref[...]
    h = jax.lax.dot_general(
        x, w1_ref[...], (((1,), (1,)), ((), ())),
        preferred_element_type=jnp.float32)
    h = jnp.maximum(h + b1_ref[...], 0.0)
    out = jax.lax.dot_general(
        h, w2_ref[...], (((1,), (1,)), ((), ())),
        preferred_element_type=jnp.float32)
    out = out + b2_ref[...]
    m = jnp.max(out, axis=1, keepdims=True)
    lse = m + jnp.log(jnp.sum(jnp.exp(out - m), axis=1, keepdims=True))
    o_ref[...] = out - lse


def kernel(x, edge_index, W1, b1, W2, b2):
    del edge_index
    n, f_in = x.shape
    hid = W1.shape[0]
    c = W2.shape[0]

    blk = 5000 if n % 5000 == 0 else n

    return pl.pallas_call(
        _mlp_logsoftmax_kernel,
        grid=(n // blk,),
        in_specs=[
            pl.BlockSpec((blk, f_in), lambda i: (i, 0)),
            pl.BlockSpec((hid, f_in), lambda i: (0, 0)),
            pl.BlockSpec((1, hid), lambda i: (0, 0)),
            pl.BlockSpec((c, hid), lambda i: (0, 0)),
            pl.BlockSpec((1, c), lambda i: (0, 0)),
        ],
        out_specs=pl.BlockSpec((blk, c), lambda i: (i, 0)),
        out_shape=jax.ShapeDtypeStruct((n, c), jnp.float32),
    )(x, W1, b1.reshape(1, hid), W2, b2.reshape(1, c))
